# baseline (speedup 1.0000x reference)
import functools

import jax
import jax.numpy as jnp
from jax import lax
from jax.experimental import pallas as pl
from jax.experimental.pallas import tpu as pltpu
from jax.experimental.pallas import tpu_sc as plsc


def _conv2(x, w, b, stride, pad):
    y = jax.lax.conv_general_dilated(
        x, w, (stride, stride), [(pad, pad), (pad, pad)],
        dimension_numbers=('NCHW', 'OIHW', 'NCHW'))
    return y + b[None, :, None, None]


def _convT2(x, w, b, stride, pad):
    wf = jnp.flip(w, (2, 3)).transpose(1, 0, 2, 3)
    co, ci = wf.shape[0], wf.shape[1]
    k2 = wf.reshape(co, ci, 2, 2, 2, 2)
    k2 = k2.transpose(3, 5, 0, 1, 2, 4).reshape(4 * co, ci, 2, 2)
    c2 = jax.lax.conv_general_dilated(
        x, k2, (1, 1), [(1, 1), (1, 1)],
        dimension_numbers=('NCHW', 'OIHW', 'NCHW'))
    B, _, HP, WP = c2.shape
    H, W = HP - 1, WP - 1
    c2 = c2.reshape(B, 2, 2, co, HP, WP)
    phases = [c2[:, py, px, :, py:py + H, px:px + W]
              for py in (0, 1) for px in (0, 1)]
    t = jnp.stack(phases, axis=-1).reshape(B, co, H, W, 2, 2)
    y = t.transpose(0, 1, 2, 4, 3, 5).reshape(B, co, 2 * H, 2 * W)
    return y + b[None, :, None, None]


def _bnorm(x, g, b):
    m = jnp.mean(x, axis=(0, 2, 3), keepdims=True)
    v = jnp.var(x, axis=(0, 2, 3), keepdims=True)
    return (x - m) / jnp.sqrt(v + 1e-5) * g[None, :, None, None] + b[None, :, None, None]


_NW = 32
_L = 16
_K = 256
_CH = 7
_UNROLL = 4


def _vq_sc_body(qx_hbm, qy_hbm, ax_hbm, ay_hbm, c_hbm, ex_hbm, ey_hbm,
                qox_hbm, qoy_hbm, loss_hbm,
                qxv, qyv, oxv, oyv, axv, ayv, cv, exv, eyv, lv, qpw):
    wid = lax.axis_index("s") * 2 + lax.axis_index("c")
    base = wid * qpw
    pltpu.sync_copy(qx_hbm.at[pl.ds(base, qpw)], qxv)
    pltpu.sync_copy(qy_hbm.at[pl.ds(base, qpw)], qyv)
    pltpu.sync_copy(ax_hbm, axv)
    pltpu.sync_copy(ay_hbm, ayv)
    pltpu.sync_copy(c_hbm, cv)
    pltpu.sync_copy(ex_hbm, exv)
    pltpu.sync_copy(ey_hbm, eyv)

    nchunk = qpw // (_L * _CH)
    loss_acc = jnp.zeros((_L,), jnp.float32)
    for chunk in range(nchunk):
        off = chunk * _CH * _L
        qxs = [qxv[pl.ds(off + k * _L, _L)] for k in range(_CH)]
        qys = [qyv[pl.ds(off + k * _L, _L)] for k in range(_CH)]
        bd0 = [jnp.full((_L,), 3.4e38, jnp.float32) for _ in range(_CH)]
        bj0 = [jnp.zeros((_L,), jnp.int32) for _ in range(_CH)]

        def body(j, carry, qxs=qxs, qys=qys):
            bd = list(carry[:_CH])
            bj = list(carry[_CH:])
            for dj in range(_UNROLL):
                jj = j * _UNROLL + dj
                jv = jnp.full((_L,), 1, jnp.int32) * jj
                a = plsc.load_gather(axv, [jv])
                b = plsc.load_gather(ayv, [jv])
                c = plsc.load_gather(cv, [jv])
                for k in range(_CH):
                    d = c + a * qxs[k] + b * qys[k]
                    upd = d < bd[k]
                    bd[k] = jnp.where(upd, d, bd[k])
                    bj[k] = jnp.where(upd, jv, bj[k])
            return tuple(bd) + tuple(bj)

        carry = lax.fori_loop(0, _K // _UNROLL, body, tuple(bd0) + tuple(bj0))
        bj = carry[_CH:]
        for k in range(_CH):
            ox = plsc.load_gather(exv, [bj[k]])
            oy = plsc.load_gather(eyv, [bj[k]])
            oxv[pl.ds(off + k * _L, _L)] = ox
            oyv[pl.ds(off + k * _L, _L)] = oy
            dx = qxs[k] - ox
            dy = qys[k] - oy
            loss_acc = loss_acc + (dx * dx + dy * dy)

    lv[...] = loss_acc
    pltpu.sync_copy(oxv, qox_hbm.at[pl.ds(base, qpw)])
    pltpu.sync_copy(oyv, qoy_hbm.at[pl.ds(base, qpw)])
    pltpu.sync_copy(lv, loss_hbm.at[wid])


@functools.partial(jax.jit, static_argnums=())
def _vq(qx, qy, emb):
    n = qx.shape[0]
    qpw = n // _NW
    ex = emb[:, 0]
    ey = emb[:, 1]
    ax = -2.0 * ex
    ay = -2.0 * ey
    c = ex * ex + ey * ey
    mesh = plsc.VectorSubcoreMesh(core_axis_name="c", subcore_axis_name="s")
    kern = functools.partial(
        pl.kernel,
        mesh=mesh,
        compiler_params=pltpu.CompilerParams(needs_layout_passes=False),
        out_type=(
            jax.ShapeDtypeStruct((n,), jnp.float32),
            jax.ShapeDtypeStruct((n,), jnp.float32),
            jax.ShapeDtypeStruct((_NW, _L), jnp.float32),
        ),
        scratch_types=[
            pltpu.VMEM((qpw,), jnp.float32),
            pltpu.VMEM((qpw,), jnp.float32),
            pltpu.VMEM((qpw,), jnp.float32),
            pltpu.VMEM((qpw,), jnp.float32),
            pltpu.VMEM((_K,), jnp.float32),
            pltpu.VMEM((_K,), jnp.float32),
            pltpu.VMEM((_K,), jnp.float32),
            pltpu.VMEM((_K,), jnp.float32),
            pltpu.VMEM((_K,), jnp.float32),
            pltpu.VMEM((_L,), jnp.float32),
        ],
    )(functools.partial(_vq_sc_body, qpw=qpw))
    return kern(qx, qy, ax, ay, c, ex, ey)


def kernel(x, params):
    p = params
    h = jax.nn.relu(_bnorm(_conv2(x, p['enc_w1'], p['enc_b1'], 2, 1), p['bn1_g'], p['bn1_b']))
    h = jax.nn.relu(_bnorm(_conv2(h, p['enc_w2'], p['enc_b2'], 2, 1), p['bn2_g'], p['bn2_b']))
    h = jax.nn.relu(_bnorm(_conv2(h, p['enc_w3'], p['enc_b3'], 2, 1), p['bn3_g'], p['bn3_b']))
    q = _conv2(h, p['pq_w'], p['pq_b'], 1, 0)
    B, C, H, W = q.shape
    qf = q.reshape(B, C, H * W)
    qx = qf[:, 0, :].reshape(-1)
    qy = qf[:, 1, :].reshape(-1)
    qox, qoy, loss_part = _vq(qx, qy, p['emb'])
    vq_loss = jnp.sum(loss_part) / (B * H * W * C) * 1.2
    quant = jnp.stack([qox.reshape(B, H * W), qoy.reshape(B, H * W)], axis=1).reshape(B, C, H, W)
    d = _conv2(quant, p['postq_w'], p['postq_b'], 1, 0)
    d = jax.nn.relu(_bnorm(_convT2(d, p['dec_w1'], p['dec_b1'], 2, 1), p['dbn1_g'], p['dbn1_b']))
    d = jax.nn.relu(_bnorm(_convT2(d, p['dec_w2'], p['dec_b2'], 2, 1), p['dbn2_g'], p['dbn2_b']))
    out = jax.nn.sigmoid(_convT2(d, p['dec_w3'], p['dec_b3'], 2, 1))
    recon = jnp.mean((x - out) ** 2)
    total = recon + vq_loss
    return (out, total)


# speedup vs baseline: 1.5797x; 1.5797x over previous
import functools

import jax
import jax.numpy as jnp
from jax import lax
from jax.experimental import pallas as pl
from jax.experimental.pallas import tpu as pltpu
from jax.experimental.pallas import tpu_sc as plsc


def _conv2(x, w, b, stride, pad):
    y = jax.lax.conv_general_dilated(
        x, w, (stride, stride), [(pad, pad), (pad, pad)],
        dimension_numbers=('NCHW', 'OIHW', 'NCHW'))
    return y + b[None, :, None, None]


def _convT2(x, w, b, stride, pad):
    k = w.shape[2]
    wf = jnp.flip(w, (2, 3)).transpose(1, 0, 2, 3)
    p = k - 1 - pad
    y = jax.lax.conv_general_dilated(
        x, wf, (1, 1), [(p, p), (p, p)], lhs_dilation=(stride, stride),
        dimension_numbers=('NCHW', 'OIHW', 'NCHW'))
    return y + b[None, :, None, None]


def _bnorm(x, g, b):
    m = jnp.mean(x, axis=(0, 2, 3), keepdims=True)
    m2 = jnp.mean(x * x, axis=(0, 2, 3), keepdims=True)
    v = jnp.maximum(m2 - m * m, 0.0)
    return (x - m) / jnp.sqrt(v + 1e-5) * g[None, :, None, None] + b[None, :, None, None]


_NW = 32
_L = 16
_K = 256
_CH = 7
_UNROLL = 4


def _vq_sc_body(qx_hbm, qy_hbm, ax_hbm, ay_hbm, c_hbm, ex_hbm, ey_hbm,
                qox_hbm, qoy_hbm, loss_hbm,
                qxv, qyv, oxv, oyv, axv, ayv, cv, exv, eyv, lv, qpw):
    wid = lax.axis_index("s") * 2 + lax.axis_index("c")
    base = wid * qpw
    pltpu.sync_copy(qx_hbm.at[pl.ds(base, qpw)], qxv)
    pltpu.sync_copy(qy_hbm.at[pl.ds(base, qpw)], qyv)
    pltpu.sync_copy(ax_hbm, axv)
    pltpu.sync_copy(ay_hbm, ayv)
    pltpu.sync_copy(c_hbm, cv)
    pltpu.sync_copy(ex_hbm, exv)
    pltpu.sync_copy(ey_hbm, eyv)

    nchunk = qpw // (_L * _CH)
    loss_acc = jnp.zeros((_L,), jnp.float32)
    for chunk in range(nchunk):
        off = chunk * _CH * _L
        qxs = [qxv[pl.ds(off + k * _L, _L)] for k in range(_CH)]
        qys = [qyv[pl.ds(off + k * _L, _L)] for k in range(_CH)]
        bd0 = [jnp.full((_L,), 3.4e38, jnp.float32) for _ in range(_CH)]
        bj0 = [jnp.zeros((_L,), jnp.int32) for _ in range(_CH)]

        def body(j, carry, qxs=qxs, qys=qys):
            bd = list(carry[:_CH])
            bj = list(carry[_CH:])
            for dj in range(_UNROLL):
                jj = j * _UNROLL + dj
                jv = jnp.full((_L,), 1, jnp.int32) * jj
                a = plsc.load_gather(axv, [jv])
                b = plsc.load_gather(ayv, [jv])
                c = plsc.load_gather(cv, [jv])
                for k in range(_CH):
                    d = c + a * qxs[k] + b * qys[k]
                    upd = d < bd[k]
                    bd[k] = jnp.where(upd, d, bd[k])
                    bj[k] = jnp.where(upd, jv, bj[k])
            return tuple(bd) + tuple(bj)

        carry = lax.fori_loop(0, _K // _UNROLL, body, tuple(bd0) + tuple(bj0))
        bj = carry[_CH:]
        for k in range(_CH):
            ox = plsc.load_gather(exv, [bj[k]])
            oy = plsc.load_gather(eyv, [bj[k]])
            oxv[pl.ds(off + k * _L, _L)] = ox
            oyv[pl.ds(off + k * _L, _L)] = oy
            dx = qxs[k] - ox
            dy = qys[k] - oy
            loss_acc = loss_acc + (dx * dx + dy * dy)

    lv[...] = loss_acc
    pltpu.sync_copy(oxv, qox_hbm.at[pl.ds(base, qpw)])
    pltpu.sync_copy(oyv, qoy_hbm.at[pl.ds(base, qpw)])
    pltpu.sync_copy(lv, loss_hbm.at[wid])


@functools.partial(jax.jit, static_argnums=())
def _vq(qx, qy, emb):
    n = qx.shape[0]
    qpw = n // _NW
    ex = emb[:, 0]
    ey = emb[:, 1]
    ax = -2.0 * ex
    ay = -2.0 * ey
    c = ex * ex + ey * ey
    mesh = plsc.VectorSubcoreMesh(core_axis_name="c", subcore_axis_name="s")
    kern = functools.partial(
        pl.kernel,
        mesh=mesh,
        compiler_params=pltpu.CompilerParams(needs_layout_passes=False),
        out_type=(
            jax.ShapeDtypeStruct((n,), jnp.float32),
            jax.ShapeDtypeStruct((n,), jnp.float32),
            jax.ShapeDtypeStruct((_NW, _L), jnp.float32),
        ),
        scratch_types=[
            pltpu.VMEM((qpw,), jnp.float32),
            pltpu.VMEM((qpw,), jnp.float32),
            pltpu.VMEM((qpw,), jnp.float32),
            pltpu.VMEM((qpw,), jnp.float32),
            pltpu.VMEM((_K,), jnp.float32),
            pltpu.VMEM((_K,), jnp.float32),
            pltpu.VMEM((_K,), jnp.float32),
            pltpu.VMEM((_K,), jnp.float32),
            pltpu.VMEM((_K,), jnp.float32),
            pltpu.VMEM((_L,), jnp.float32),
        ],
    )(functools.partial(_vq_sc_body, qpw=qpw))
    return kern(qx, qy, ax, ay, c, ex, ey)


def kernel(x, params):
    p = params
    h = jax.nn.relu(_bnorm(_conv2(x, p['enc_w1'], p['enc_b1'], 2, 1), p['bn1_g'], p['bn1_b']))
    h = jax.nn.relu(_bnorm(_conv2(h, p['enc_w2'], p['enc_b2'], 2, 1), p['bn2_g'], p['bn2_b']))
    h = jax.nn.relu(_bnorm(_conv2(h, p['enc_w3'], p['enc_b3'], 2, 1), p['bn3_g'], p['bn3_b']))
    q = _conv2(h, p['pq_w'], p['pq_b'], 1, 0)
    B, C, H, W = q.shape
    qf = q.reshape(B, C, H * W)
    qx = qf[:, 0, :].reshape(-1)
    qy = qf[:, 1, :].reshape(-1)
    qox, qoy, loss_part = _vq(qx, qy, p['emb'])
    vq_loss = jnp.sum(loss_part) / (B * H * W * C) * 1.2
    quant = jnp.stack([qox.reshape(B, H * W), qoy.reshape(B, H * W)], axis=1).reshape(B, C, H, W)
    d = _conv2(quant, p['postq_w'], p['postq_b'], 1, 0)
    d = jax.nn.relu(_bnorm(_convT2(d, p['dec_w1'], p['dec_b1'], 2, 1), p['dbn1_g'], p['dbn1_b']))
    d = jax.nn.relu(_bnorm(_convT2(d, p['dec_w2'], p['dec_b2'], 2, 1), p['dbn2_g'], p['dbn2_b']))
    out = jax.nn.sigmoid(_convT2(d, p['dec_w3'], p['dec_b3'], 2, 1))
    recon = jnp.mean((x - out) ** 2)
    total = recon + vq_loss
    return (out, total)
